# baseline (speedup 1.0000x reference)
import jax
import jax.numpy as jnp
from jax.experimental import pallas as pl
from jax.experimental.pallas import tpu as pltpu

_EPS = 1e-06
_BRH = 64
_BB = 1


def _ln_kernel(act_ref, x_ref, w_ref, b_ref, o_ref):
    for bb in range(x_ref.shape[0]):
        _ln_one(act_ref, x_ref, w_ref, b_ref, o_ref, bb)


def _ln_one(act_ref, x_ref, w_ref, b_ref, o_ref, bb):
    xb = x_ref[bb]
    br, c, wdim = xb.shape
    u = jnp.mean(xb, axis=1, keepdims=True)
    xc = xb - u
    s = jnp.mean(xc * xc, axis=1, keepdims=True)
    act = act_ref[bb, 0].astype(jnp.float32)
    nh8 = act.shape[0]
    row_h = jax.lax.broadcasted_iota(jnp.int32, (br, nh8), 0) // 8
    col_h = jax.lax.broadcasted_iota(jnp.int32, (br, nh8), 1)
    eh = (row_h == col_h).astype(jnp.float32)
    row_w = jax.lax.broadcasted_iota(jnp.int32, (16, wdim), 1) // 8
    col_w = jax.lax.broadcasted_iota(jnp.int32, (16, wdim), 0)
    ew = (row_w == col_w).astype(jnp.float32)
    m = jnp.dot(jnp.dot(eh, act, preferred_element_type=jnp.float32), ew,
                preferred_element_type=jnp.float32)

    mm = m[:, None, :]
    inv = jax.lax.rsqrt(s + _EPS)
    o_ref[bb] = ((xc * inv) * w_ref[...] + b_ref[...]) * mm


def kernel(x, active, weight, bias):
    B, H, W, C = x.shape
    nh8 = _BRH // 8
    xt = jnp.transpose(x, (0, 1, 3, 2))
    wl = jnp.broadcast_to(weight[:, None], (C, W))
    bl = jnp.broadcast_to(bias[:, None], (C, W))
    grid = (B // _BB, H // _BRH)
    out = pl.pallas_call(
        _ln_kernel,
        grid=grid,
        in_specs=[
            pl.BlockSpec((_BB, 1, nh8, 16), lambda b, i: (b, 0, i, 0)),
            pl.BlockSpec((_BB, _BRH, C, W), lambda b, i: (b, i, 0, 0)),
            pl.BlockSpec((C, W), lambda b, i: (0, 0)),
            pl.BlockSpec((C, W), lambda b, i: (0, 0)),
        ],
        out_specs=pl.BlockSpec((_BB, _BRH, C, W), lambda b, i: (b, i, 0, 0)),
        out_shape=jax.ShapeDtypeStruct((B, H, C, W), x.dtype),
        compiler_params=pltpu.CompilerParams(
            dimension_semantics=("parallel", "parallel")),
    )(active, xt, wl, bl)
    return jnp.transpose(out, (0, 1, 3, 2))


# speedup vs baseline: 1.1290x; 1.1290x over previous
import jax
import jax.numpy as jnp
from jax.experimental import pallas as pl
from jax.experimental.pallas import tpu as pltpu

_EPS = 1e-06
_BRH = 128
_BB = 1


def _ln_kernel(act_ref, x_ref, w_ref, b_ref, o_ref):
    for bb in range(x_ref.shape[0]):
        _ln_one(act_ref, x_ref, w_ref, b_ref, o_ref, bb)


def _ln_one(act_ref, x_ref, w_ref, b_ref, o_ref, bb):
    xb = x_ref[bb]
    br, c, wdim = xb.shape
    u = jnp.mean(xb, axis=1, keepdims=True)
    xc = xb - u
    s = jnp.mean(xc * xc, axis=1, keepdims=True)
    act = act_ref[bb, 0].astype(jnp.float32)
    nh8 = act.shape[0]
    row_h = jax.lax.broadcasted_iota(jnp.int32, (br, nh8), 0) // 8
    col_h = jax.lax.broadcasted_iota(jnp.int32, (br, nh8), 1)
    eh = (row_h == col_h).astype(jnp.float32)
    row_w = jax.lax.broadcasted_iota(jnp.int32, (16, wdim), 1) // 8
    col_w = jax.lax.broadcasted_iota(jnp.int32, (16, wdim), 0)
    ew = (row_w == col_w).astype(jnp.float32)
    m = jnp.dot(jnp.dot(eh, act, preferred_element_type=jnp.float32), ew,
                preferred_element_type=jnp.float32)

    mm = m[:, None, :]
    inv = jax.lax.rsqrt(s + _EPS)
    o_ref[bb] = ((xc * inv) * w_ref[...] + b_ref[...]) * mm


def kernel(x, active, weight, bias):
    B, H, W, C = x.shape
    nh8 = _BRH // 8
    xt = jnp.transpose(x, (0, 1, 3, 2))
    wl = jnp.broadcast_to(weight[:, None], (C, W))
    bl = jnp.broadcast_to(bias[:, None], (C, W))
    grid = (B // _BB, H // _BRH)
    out = pl.pallas_call(
        _ln_kernel,
        grid=grid,
        in_specs=[
            pl.BlockSpec((_BB, 1, nh8, 16), lambda b, i: (b, 0, i, 0)),
            pl.BlockSpec((_BB, _BRH, C, W), lambda b, i: (b, i, 0, 0)),
            pl.BlockSpec((C, W), lambda b, i: (0, 0)),
            pl.BlockSpec((C, W), lambda b, i: (0, 0)),
        ],
        out_specs=pl.BlockSpec((_BB, _BRH, C, W), lambda b, i: (b, i, 0, 0)),
        out_shape=jax.ShapeDtypeStruct((B, H, C, W), x.dtype),
        compiler_params=pltpu.CompilerParams(
            dimension_semantics=("parallel", "parallel")),
    )(active, xt, wl, bl)
    return jnp.transpose(out, (0, 1, 3, 2))
